# baseline (speedup 1.0000x reference)
import functools
from math import sqrt

import numpy as np
import jax
import jax.numpy as jnp
from jax.experimental import pallas as pl
from jax.experimental.pallas import tpu as pltpu

_FACTOR = 5

_COUNTS_CACHE = {}


def _rotl32(x, r):
    return ((x << np.uint32(r)) | (x >> np.uint32(32 - r))).astype(np.uint32)


def _threefry2x32(k0, k1, x0, x1):
    rotations = ((13, 15, 26, 6), (17, 29, 16, 24))
    ks = (np.uint32(k0), np.uint32(k1),
          np.uint32(k0) ^ np.uint32(k1) ^ np.uint32(0x1BD11BDA))
    x0 = (x0 + ks[0]).astype(np.uint32)
    x1 = (x1 + ks[1]).astype(np.uint32)
    for i in range(5):
        for r in rotations[i % 2]:
            x0 = (x0 + x1).astype(np.uint32)
            x1 = _rotl32(x1, r)
            x1 = x0 ^ x1
        x0 = (x0 + ks[(i + 1) % 3]).astype(np.uint32)
        x1 = (x1 + ks[(i + 2) % 3] + np.uint32(i + 1)).astype(np.uint32)
    return x0, x1


def _np_uniform(seed, shape):
    n = int(np.prod(shape))
    k0 = np.uint32(np.uint64(seed) >> np.uint64(32))
    k1 = np.uint32(np.uint64(seed) & np.uint64(0xFFFFFFFF))
    b0, b1 = _threefry2x32(k0, k1, np.zeros(n, np.uint32),
                           np.arange(n, dtype=np.uint32))
    bits = b0 ^ b1
    f = ((bits >> np.uint32(9)) | np.uint32(0x3F800000)).view(np.float32)
    return (f - np.float32(1.0)).reshape(shape)


def _sample_counts_t(L_Q, L_K, U_part):
    cache_key = (L_Q, L_K, U_part)
    if cache_key not in _COUNTS_CACHE:
        idx = (_np_uniform(42, (L_Q, U_part)) * L_K).astype(np.int32)
        counts = np.zeros((L_Q, L_K), np.float32)
        np.add.at(counts, (np.arange(L_Q)[:, None], idx), 1.0)
        _COUNTS_CACHE[cache_key] = np.ascontiguousarray(counts.T)
    return jnp.asarray(_COUNTS_CACHE[cache_key])


def _stats_topk_body(ct_ref, q_ref, k_ref, r_ref, m_scr,
                     *, U_part, n_top, blk):
    s = pl.program_id(0)
    H, L, D = q_ref.shape
    nblk = L // blk

    @pl.when(s < H)
    def _stats():
        q = q_ref[s]
        k = k_ref[s]
        for j in range(nblk):
            q_blk = q[j * blk:(j + 1) * blk, :]
            st = jax.lax.dot_general(k, q_blk, (((1,), (1,)), ((), ())),
                                     preferred_element_type=jnp.float32)
            ct = ct_ref[:, j * blk:(j + 1) * blk]
            mx = jnp.max(jnp.where(ct > 0.0, st, -jnp.inf), axis=0,
                         keepdims=True)
            sm = jnp.sum(st * ct, axis=0, keepdims=True)
            m_scr[pl.ds(s, 1), j * blk:(j + 1) * blk] = mx - sm * (1.0 / U_part)

    @pl.when(s == H)
    def _select():
        m = m_scr[...]
        lane = jax.lax.broadcasted_iota(jnp.int32, (H, L), 1)

        def sel_body(i, carry):
            m_cur, rank = carry
            cur = jnp.max(m_cur, axis=1, keepdims=True)
            j = jnp.min(jnp.where(m_cur == cur, lane, L), axis=1,
                        keepdims=True)
            hit = lane == j
            rank = jnp.where(hit, i, rank)
            m_cur = jnp.where(hit, -jnp.inf, m_cur)
            return m_cur, rank

        _, rank = jax.lax.fori_loop(
            0, n_top, sel_body, (m, jnp.full((H, L), -1, jnp.int32)))
        r_ref[...] = rank


def _attn_body(r_ref, q_ref, k_ref, v_ref, o_ref, *, n_top, scale):
    rank_row = r_ref[0]
    L = rank_row.shape[1]
    rowio = jax.lax.broadcasted_iota(jnp.int32, (n_top, L), 0)
    oselt = (rowio == rank_row).astype(jnp.float32)

    q = q_ref[0]
    k = k_ref[0]
    v = v_ref[0]
    qsel = jax.lax.dot_general(oselt, q, (((1,), (0,)), ((), ())),
                               preferred_element_type=jnp.float32)
    scores = jax.lax.dot_general(qsel, k, (((1,), (1,)), ((), ())),
                                 preferred_element_type=jnp.float32) * scale
    scores = scores - jnp.max(scores, axis=1, keepdims=True)
    e = jnp.exp(scores)
    p = e / jnp.sum(e, axis=1, keepdims=True)
    upd = jnp.dot(p, v, preferred_element_type=jnp.float32)
    meanv = jnp.mean(v, axis=0, keepdims=True)
    scattered = jax.lax.dot_general(oselt, upd, (((0,), (0,)), ((), ())),
                                    preferred_element_type=jnp.float32)
    selcol = jax.lax.dot_general(oselt, jnp.ones((n_top, 1), jnp.float32),
                                 (((0,), (0,)), ((), ())),
                                 preferred_element_type=jnp.float32)
    o_ref[0] = scattered + (1.0 - selcol) * meanv


@functools.partial(jax.jit, static_argnames=("U_part", "n_top"))
def _impl(queries, keys, values, counts_t, U_part, n_top):
    B, L, H, D = queries.shape
    L_K = keys.shape[1]
    q3 = jnp.transpose(queries[0], (1, 0, 2))
    k3 = jnp.transpose(keys[0], (1, 0, 2))
    v3 = jnp.transpose(values[0], (1, 0, 2))

    BLK = 256
    rank = pl.pallas_call(
        functools.partial(_stats_topk_body, U_part=U_part, n_top=n_top,
                          blk=BLK),
        grid=(H + 1,),
        in_specs=[
            pl.BlockSpec((L_K, L), lambda s: (0, 0)),
            pl.BlockSpec((H, L, D), lambda s: (0, 0, 0)),
            pl.BlockSpec((H, L_K, D), lambda s: (0, 0, 0)),
        ],
        out_specs=pl.BlockSpec((H, L), lambda s: (0, 0)),
        out_shape=jax.ShapeDtypeStruct((H, L), jnp.int32),
        scratch_shapes=[pltpu.VMEM((H, L), jnp.float32)],
    )(counts_t, q3, k3)

    out = pl.pallas_call(
        functools.partial(_attn_body, n_top=n_top, scale=1.0 / sqrt(D)),
        grid=(H,),
        in_specs=[
            pl.BlockSpec((1, 1, L), lambda h: (h, 0, 0)),
            pl.BlockSpec((1, L, D), lambda h: (h, 0, 0)),
            pl.BlockSpec((1, L_K, D), lambda h: (h, 0, 0)),
            pl.BlockSpec((1, L_K, D), lambda h: (h, 0, 0)),
        ],
        out_specs=pl.BlockSpec((1, L, D), lambda h: (h, 0, 0)),
        out_shape=jax.ShapeDtypeStruct((H, L, D), jnp.float32),
    )(jnp.reshape(rank, (H, 1, L)), q3, k3, v3)

    return out[None]


def kernel(queries, keys, values, attn_mask):
    B, L, H, D = queries.shape
    L_K = keys.shape[1]
    U_part = min(int(_FACTOR * np.ceil(np.log(L_K))), L_K)
    n_top = min(int(_FACTOR * np.ceil(np.log(L))), L)
    counts_t = _sample_counts_t(L, L_K, U_part)
    return _impl(queries, keys, values, counts_t, U_part, n_top)


# speedup vs baseline: 1.0897x; 1.0897x over previous
import functools
from math import sqrt

import numpy as np
import jax
import jax.numpy as jnp
from jax.experimental import pallas as pl
from jax.experimental.pallas import tpu as pltpu

_FACTOR = 5

_COUNTS_CACHE = {}


def _rotl32(x, r):
    return ((x << np.uint32(r)) | (x >> np.uint32(32 - r))).astype(np.uint32)


def _threefry2x32(k0, k1, x0, x1):
    rotations = ((13, 15, 26, 6), (17, 29, 16, 24))
    ks = (np.uint32(k0), np.uint32(k1),
          np.uint32(k0) ^ np.uint32(k1) ^ np.uint32(0x1BD11BDA))
    x0 = (x0 + ks[0]).astype(np.uint32)
    x1 = (x1 + ks[1]).astype(np.uint32)
    for i in range(5):
        for r in rotations[i % 2]:
            x0 = (x0 + x1).astype(np.uint32)
            x1 = _rotl32(x1, r)
            x1 = x0 ^ x1
        x0 = (x0 + ks[(i + 1) % 3]).astype(np.uint32)
        x1 = (x1 + ks[(i + 2) % 3] + np.uint32(i + 1)).astype(np.uint32)
    return x0, x1


def _np_uniform(seed, shape):
    n = int(np.prod(shape))
    k0 = np.uint32(np.uint64(seed) >> np.uint64(32))
    k1 = np.uint32(np.uint64(seed) & np.uint64(0xFFFFFFFF))
    b0, b1 = _threefry2x32(k0, k1, np.zeros(n, np.uint32),
                           np.arange(n, dtype=np.uint32))
    bits = b0 ^ b1
    f = ((bits >> np.uint32(9)) | np.uint32(0x3F800000)).view(np.float32)
    return (f - np.float32(1.0)).reshape(shape)


def _sample_counts_t(L_Q, L_K, U_part):
    cache_key = (L_Q, L_K, U_part)
    if cache_key not in _COUNTS_CACHE:
        idx = (_np_uniform(42, (L_Q, U_part)) * L_K).astype(np.int32)
        counts = np.zeros((L_Q, L_K), np.float32)
        np.add.at(counts, (np.arange(L_Q)[:, None], idx), 1.0)
        _COUNTS_CACHE[cache_key] = np.ascontiguousarray(counts.T)
    return jnp.asarray(_COUNTS_CACHE[cache_key])


def _stats_topk_body(ct_ref, q_ref, k_ref, r_ref, m_scr,
                     *, H, U_part, n_top, blk, D):
    s = pl.program_id(0)
    L = q_ref.shape[0]
    nblk = L // blk
    npair = H // 2

    @pl.when(s < npair)
    def _stats():
        q01 = q_ref[...]
        k01 = k_ref[...]
        for t in range(2):
            q = q01[:, t * D:(t + 1) * D]
            k = k01[:, t * D:(t + 1) * D]
            for j in range(nblk):
                q_blk = q[j * blk:(j + 1) * blk, :]
                st = jax.lax.dot_general(k, q_blk, (((1,), (1,)), ((), ())),
                                         preferred_element_type=jnp.float32)
                ct = ct_ref[:, j * blk:(j + 1) * blk]
                mx = jnp.max(jnp.where(ct > 0.0, st, -jnp.inf), axis=0,
                             keepdims=True)
                sm = jnp.sum(st * ct, axis=0, keepdims=True)
                m_scr[pl.ds(2 * s + t, 1), j * blk:(j + 1) * blk] = (
                    mx - sm * (1.0 / U_part))

    @pl.when(s == npair)
    def _select():
        m = m_scr[...]
        lane = jax.lax.broadcasted_iota(jnp.int32, (H, L), 1)

        def sel_body(i, carry):
            m_cur, rank = carry
            cur = jnp.max(m_cur, axis=1, keepdims=True)
            j = jnp.min(jnp.where(m_cur == cur, lane, L), axis=1,
                        keepdims=True)
            hit = lane == j
            rank = jnp.where(hit, i, rank)
            m_cur = jnp.where(hit, -jnp.inf, m_cur)
            return m_cur, rank

        _, rank = jax.lax.fori_loop(
            0, n_top, sel_body, (m, jnp.full((H, L), -1, jnp.int32)))
        r_ref[...] = rank


def _attn_body(r_ref, q_ref, k_ref, v_ref, o_ref, *, n_top, scale, D):
    r01 = r_ref[0]
    L = r01.shape[1]
    rowio = jax.lax.broadcasted_iota(jnp.int32, (n_top, L), 0)
    ones_col = jnp.ones((n_top, 1), jnp.float32)
    q01 = q_ref[...]
    k01 = k_ref[...]
    v01 = v_ref[...]
    for t in range(2):
        rank_row = r01[t:t + 1, :]
        oselt = (rowio == rank_row).astype(jnp.float32)
        q = q01[:, t * D:(t + 1) * D]
        k = k01[:, t * D:(t + 1) * D]
        v = v01[:, t * D:(t + 1) * D]
        qsel = jax.lax.dot_general(oselt, q, (((1,), (0,)), ((), ())),
                                   preferred_element_type=jnp.float32)
        scores = jax.lax.dot_general(qsel, k, (((1,), (1,)), ((), ())),
                                     preferred_element_type=jnp.float32) * scale
        scores = scores - jnp.max(scores, axis=1, keepdims=True)
        e = jnp.exp(scores)
        p = e / jnp.sum(e, axis=1, keepdims=True)
        upd = jnp.dot(p, v, preferred_element_type=jnp.float32)
        meanv = jnp.mean(v, axis=0, keepdims=True)
        scattered = jax.lax.dot_general(oselt, upd, (((0,), (0,)), ((), ())),
                                        preferred_element_type=jnp.float32)
        selcol = jax.lax.dot_general(oselt, ones_col, (((0,), (0,)), ((), ())),
                                     preferred_element_type=jnp.float32)
        o_ref[t] = scattered + (1.0 - selcol) * meanv


@functools.partial(jax.jit, static_argnames=("U_part", "n_top"))
def _impl(queries, keys, values, counts_t, U_part, n_top):
    B, L, H, D = queries.shape
    L_K = keys.shape[1]
    q2 = jnp.reshape(queries[0], (L, H * D))
    k2 = jnp.reshape(keys[0], (L_K, H * D))
    v2 = jnp.reshape(values[0], (L_K, H * D))

    BLK = 256
    npair = H // 2
    rank = pl.pallas_call(
        functools.partial(_stats_topk_body, H=H, U_part=U_part, n_top=n_top,
                          blk=BLK, D=D),
        grid=(npair + 1,),
        in_specs=[
            pl.BlockSpec((L_K, L), lambda s: (0, 0)),
            pl.BlockSpec((L, 2 * D),
                         lambda s, _n=npair - 1: (0, jnp.minimum(s, _n))),
            pl.BlockSpec((L_K, 2 * D),
                         lambda s, _n=npair - 1: (0, jnp.minimum(s, _n))),
        ],
        out_specs=pl.BlockSpec((H, L), lambda s: (0, 0)),
        out_shape=jax.ShapeDtypeStruct((H, L), jnp.int32),
        scratch_shapes=[pltpu.VMEM((H, L), jnp.float32)],
    )(counts_t, q2, k2)

    out = pl.pallas_call(
        functools.partial(_attn_body, n_top=n_top, scale=1.0 / sqrt(D), D=D),
        grid=(npair,),
        in_specs=[
            pl.BlockSpec((1, 2, L), lambda g: (g, 0, 0)),
            pl.BlockSpec((L, 2 * D), lambda g: (0, g)),
            pl.BlockSpec((L_K, 2 * D), lambda g: (0, g)),
            pl.BlockSpec((L_K, 2 * D), lambda g: (0, g)),
        ],
        out_specs=pl.BlockSpec((2, L, D), lambda g: (g, 0, 0)),
        out_shape=jax.ShapeDtypeStruct((H, L, D), jnp.float32),
    )(jnp.reshape(rank, (npair, 2, L)), q2, k2, v2)

    return out[None]


def kernel(queries, keys, values, attn_mask):
    B, L, H, D = queries.shape
    L_K = keys.shape[1]
    U_part = min(int(_FACTOR * np.ceil(np.log(L_K))), L_K)
    n_top = min(int(_FACTOR * np.ceil(np.log(L))), L)
    counts_t = _sample_counts_t(L, L_K, U_part)
    return _impl(queries, keys, values, counts_t, U_part, n_top)


# speedup vs baseline: 1.1201x; 1.0279x over previous
import functools
from math import sqrt

import numpy as np
import jax
import jax.numpy as jnp
from jax.experimental import pallas as pl
from jax.experimental.pallas import tpu as pltpu

_FACTOR = 5

_COUNTS_CACHE = {}


def _rotl32(x, r):
    return ((x << np.uint32(r)) | (x >> np.uint32(32 - r))).astype(np.uint32)


def _threefry2x32(k0, k1, x0, x1):
    rotations = ((13, 15, 26, 6), (17, 29, 16, 24))
    ks = (np.uint32(k0), np.uint32(k1),
          np.uint32(k0) ^ np.uint32(k1) ^ np.uint32(0x1BD11BDA))
    x0 = (x0 + ks[0]).astype(np.uint32)
    x1 = (x1 + ks[1]).astype(np.uint32)
    for i in range(5):
        for r in rotations[i % 2]:
            x0 = (x0 + x1).astype(np.uint32)
            x1 = _rotl32(x1, r)
            x1 = x0 ^ x1
        x0 = (x0 + ks[(i + 1) % 3]).astype(np.uint32)
        x1 = (x1 + ks[(i + 2) % 3] + np.uint32(i + 1)).astype(np.uint32)
    return x0, x1


def _np_uniform(seed, shape):
    n = int(np.prod(shape))
    k0 = np.uint32(np.uint64(seed) >> np.uint64(32))
    k1 = np.uint32(np.uint64(seed) & np.uint64(0xFFFFFFFF))
    b0, b1 = _threefry2x32(k0, k1, np.zeros(n, np.uint32),
                           np.arange(n, dtype=np.uint32))
    bits = b0 ^ b1
    f = ((bits >> np.uint32(9)) | np.uint32(0x3F800000)).view(np.float32)
    return (f - np.float32(1.0)).reshape(shape)


def _sample_counts_t(L_Q, L_K, U_part):
    cache_key = (L_Q, L_K, U_part)
    if cache_key not in _COUNTS_CACHE:
        idx = (_np_uniform(42, (L_Q, U_part)) * L_K).astype(np.int32)
        counts = np.zeros((L_Q, L_K), np.float32)
        np.add.at(counts, (np.arange(L_Q)[:, None], idx), 1.0)
        _COUNTS_CACHE[cache_key] = np.ascontiguousarray(counts.T)
    return jnp.asarray(_COUNTS_CACHE[cache_key])


def _fused_body(ct_ref, q_ref, k_ref, v_ref, o_ref, m_scr, r_scr,
                *, H, U_part, n_top, scale, blk, D):
    s = pl.program_id(0)
    L = q_ref.shape[0]
    nblk = L // blk
    npair = H // 2

    @pl.when(s < npair)
    def _stats():
        q01 = q_ref[...]
        k01 = k_ref[...]
        for t in range(2):
            q = q01[:, t * D:(t + 1) * D]
            k = k01[:, t * D:(t + 1) * D]
            for j in range(nblk):
                q_blk = q[j * blk:(j + 1) * blk, :]
                st = jax.lax.dot_general(k, q_blk, (((1,), (1,)), ((), ())),
                                         preferred_element_type=jnp.float32)
                ct = ct_ref[:, j * blk:(j + 1) * blk]
                mx = jnp.max(jnp.where(ct > 0.0, st, -jnp.inf), axis=0,
                             keepdims=True)
                sm = jnp.sum(st * ct, axis=0, keepdims=True)
                m_scr[pl.ds(2 * s + t, 1), j * blk:(j + 1) * blk] = (
                    mx - sm * (1.0 / U_part))

    @pl.when(s == npair)
    def _select():
        m = m_scr[...]
        lane = jax.lax.broadcasted_iota(jnp.int32, (H, L), 1)

        def sel_body(i, carry):
            m_cur, rank = carry
            cur = jnp.max(m_cur, axis=1, keepdims=True)
            j = jnp.min(jnp.where(m_cur == cur, lane, L), axis=1,
                        keepdims=True)
            hit = lane == j
            rank = jnp.where(hit, i, rank)
            m_cur = jnp.where(hit, -jnp.inf, m_cur)
            return m_cur, rank

        _, rank = jax.lax.fori_loop(
            0, n_top, sel_body, (m, jnp.full((H, L), -1, jnp.int32)))
        for p in range(npair):
            r_scr[p] = rank[2 * p:2 * p + 2, :]

    @pl.when(s > npair)
    def _attend():
        g = s - npair - 1
        rowio = jax.lax.broadcasted_iota(jnp.int32, (n_top, L), 0)
        ones_col = jnp.ones((n_top, 1), jnp.float32)
        q01 = q_ref[...]
        k01 = k_ref[...]
        v01 = v_ref[...]
        r01 = r_scr[g]
        for t in range(2):
            rank_row = r01[t:t + 1, :]
            oselt = (rowio == rank_row).astype(jnp.float32)
            q = q01[:, t * D:(t + 1) * D]
            k = k01[:, t * D:(t + 1) * D]
            v = v01[:, t * D:(t + 1) * D]
            qsel = jax.lax.dot_general(oselt, q, (((1,), (0,)), ((), ())),
                                       preferred_element_type=jnp.float32)
            scores = jax.lax.dot_general(qsel, k, (((1,), (1,)), ((), ())),
                                         preferred_element_type=jnp.float32)
            scores = scores * scale
            scores = scores - jnp.max(scores, axis=1, keepdims=True)
            e = jnp.exp(scores)
            p = e / jnp.sum(e, axis=1, keepdims=True)
            upd = jnp.dot(p, v, preferred_element_type=jnp.float32)
            meanv = jnp.mean(v, axis=0, keepdims=True)
            scattered = jax.lax.dot_general(oselt, upd,
                                            (((0,), (0,)), ((), ())),
                                            preferred_element_type=jnp.float32)
            selcol = jax.lax.dot_general(oselt, ones_col,
                                         (((0,), (0,)), ((), ())),
                                         preferred_element_type=jnp.float32)
            o_ref[t] = scattered + (1.0 - selcol) * meanv


@functools.partial(jax.jit, static_argnames=("U_part", "n_top"))
def _impl(queries, keys, values, counts_t, U_part, n_top):
    B, L, H, D = queries.shape
    L_K = keys.shape[1]
    q2 = jnp.reshape(queries[0], (L, H * D))
    k2 = jnp.reshape(keys[0], (L_K, H * D))
    v2 = jnp.reshape(values[0], (L_K, H * D))

    BLK = 256
    npair = H // 2

    def _qk_idx(s, _n=npair):
        return (0, jnp.where(s < _n, jnp.minimum(s, _n - 1),
                             jnp.maximum(s - _n - 1, 0)))

    def _out_idx(s, _n=npair):
        return (jnp.clip(s - _n - 1, 0, _n - 1), 0, 0)

    out = pl.pallas_call(
        functools.partial(_fused_body, H=H, U_part=U_part, n_top=n_top,
                          scale=1.0 / sqrt(D), blk=BLK, D=D),
        grid=(2 * npair + 1,),
        in_specs=[
            pl.BlockSpec((L_K, L), lambda s: (0, 0)),
            pl.BlockSpec((L, 2 * D), _qk_idx),
            pl.BlockSpec((L_K, 2 * D), _qk_idx),
            pl.BlockSpec((L_K, 2 * D), _qk_idx),
        ],
        out_specs=pl.BlockSpec((2, L, D), _out_idx),
        out_shape=jax.ShapeDtypeStruct((H, L, D), jnp.float32),
        scratch_shapes=[pltpu.VMEM((H, L), jnp.float32),
                        pltpu.VMEM((H // 2, 2, L), jnp.int32)],
    )(counts_t, q2, k2, v2)

    return out[None]


def kernel(queries, keys, values, attn_mask):
    B, L, H, D = queries.shape
    L_K = keys.shape[1]
    U_part = min(int(_FACTOR * np.ceil(np.log(L_K))), L_K)
    n_top = min(int(_FACTOR * np.ceil(np.log(L))), L)
    counts_t = _sample_counts_t(L, L_K, U_part)
    return _impl(queries, keys, values, counts_t, U_part, n_top)
